# baseline (speedup 1.0000x reference)
import functools

import jax
import jax.numpy as jnp
from jax import lax
from jax.experimental import pallas as pl
from jax.experimental.pallas import tpu as pltpu
from jax.experimental.pallas import tpu_sc as plsc

N = 10000
E = 320000
D_IN = 128
D_HID = 32
D_OUT = 16

NC = 2
NS = 16
NW = NC * NS

C = 128
NB = 8
TCH = 80
NCHUNKS = NW * TCH
EPAD = NCHUNKS * C
NPAD = 10240
RPT = NPAD // NS
NGRP = TCH // NB


def _mesh():
    return plsc.VectorSubcoreMesh(core_axis_name="c", subcore_axis_name="s")


def _sc_degrees(src2d, dst2d, zrow):
    @functools.partial(
        pl.kernel,
        out_type=jax.ShapeDtypeStruct((2, NC, NPAD), jnp.float32),
        mesh=_mesh(),
        scratch_types=[
            pltpu.VMEM((TCH, C), jnp.int32),
            pltpu.VMEM((C,), jnp.float32),
            pltpu.VMEM_SHARED((NPAD,), jnp.float32),
            pltpu.VMEM_SHARED((NPAD,), jnp.float32),
        ],
    )
    def deg_kernel(src_hbm, dst_hbm, z_hbm, out_hbm, idx_v, ones_v, acc_out,
                   acc_in):
        c = lax.axis_index("c")
        s = lax.axis_index("s")
        wid = c * NS + s
        for i in range(C // 16):
            ones_v[pl.ds(i * 16, 16)] = jnp.ones((16,), jnp.float32)
        pltpu.sync_copy(z_hbm.at[pl.ds(s * RPT, RPT)],
                        acc_out.at[pl.ds(s * RPT, RPT)])
        pltpu.sync_copy(z_hbm.at[pl.ds(s * RPT, RPT)],
                        acc_in.at[pl.ds(s * RPT, RPT)])
        plsc.subcore_barrier()

        for idx_hbm, acc in ((src_hbm, acc_out), (dst_hbm, acc_in)):
            pltpu.sync_copy(idx_hbm.at[pl.ds(wid * TCH, TCH)], idx_v)

            @pl.loop(0, TCH)
            def _chunk(j, acc=acc):
                pltpu.sync_copy(ones_v, acc.at[idx_v.at[j]], add=True)

        plsc.subcore_barrier()
        pltpu.sync_copy(acc_out.at[pl.ds(s * RPT, RPT)],
                        out_hbm.at[0, c, pl.ds(s * RPT, RPT)])
        pltpu.sync_copy(acc_in.at[pl.ds(s * RPT, RPT)],
                        out_hbm.at[1, c, pl.ds(s * RPT, RPT)])

    return deg_kernel(src2d, dst2d, zrow)


def _sc_propagate(h, src2d, dst2d, zero_acc, d):
    @functools.partial(
        pl.kernel,
        out_type=jax.ShapeDtypeStruct((NC, NPAD, d), jnp.float32),
        mesh=_mesh(),
        compiler_params=pltpu.CompilerParams(use_tc_tiling_on_sc=False),
        scratch_types=[
            pltpu.VMEM((TCH, C), jnp.int32),
            pltpu.VMEM((TCH, C), jnp.int32),
            pltpu.VMEM((NB, C, d), jnp.float32),
            pltpu.VMEM_SHARED((NPAD, d), jnp.float32),
            pltpu.SemaphoreType.DMA((NB,)),
            pltpu.SemaphoreType.DMA((NB,)),
        ],
    )
    def prop_kernel(h_hbm, src_hbm, dst_hbm, z_hbm, out_hbm, src_idx, dst_idx,
                    rows, acc, gsem, ssem):
        c = lax.axis_index("c")
        s = lax.axis_index("s")
        wid = c * NS + s
        pltpu.sync_copy(src_hbm.at[pl.ds(wid * TCH, TCH)], src_idx)
        pltpu.sync_copy(dst_hbm.at[pl.ds(wid * TCH, TCH)], dst_idx)

        def gstart(j, bb):
            pltpu.async_copy(h_hbm.at[src_idx.at[j]], rows.at[bb], gsem.at[bb])

        def gwait(j, bb):
            pltpu.make_async_copy(h_hbm.at[src_idx.at[j]], rows.at[bb],
                                  gsem.at[bb]).wait()

        def sstart(j, bb):
            pltpu.async_copy(rows.at[bb], acc.at[dst_idx.at[j]], ssem.at[bb],
                             add=True)

        def swait(j, bb):
            pltpu.make_async_copy(rows.at[bb], acc.at[dst_idx.at[j]],
                                  ssem.at[bb]).wait()

        for b in range(NB):
            gstart(b, b)
        pltpu.sync_copy(z_hbm.at[pl.ds(s * RPT, RPT)],
                        acc.at[pl.ds(s * RPT, RPT)])
        plsc.subcore_barrier()

        @pl.loop(0, NGRP - 1)
        def _grp(g):
            for b in range(NB):
                j = g * NB + b
                gwait(j, b)
                sstart(j, b)
            for b in range(NB):
                j = g * NB + b
                swait(j, b)
                gstart(j + NB, b)

        for b in range(NB):
            j = (NGRP - 1) * NB + b
            gwait(j, b)
            sstart(j, b)
        for b in range(NB):
            swait((NGRP - 1) * NB + b, b)

        plsc.subcore_barrier()
        pltpu.sync_copy(acc.at[pl.ds(s * RPT, RPT)],
                        out_hbm.at[c, pl.ds(s * RPT, RPT)])

    return prop_kernel(h, src2d, dst2d, zero_acc)


def _dinv(deg):
    return jnp.where(deg > 0, lax.rsqrt(jnp.maximum(deg, 1e-12)), 0.0)


def _tc_encode1(degp4, x, w1):
    def body(degp_ref, x_ref, w_ref, h_ref, dsrc_ref, ddst_ref):
        dsrc = _dinv(degp_ref[0, 0] + degp_ref[0, 1])
        ddst = _dinv(degp_ref[1, 0] + degp_ref[1, 1])
        dsrc_ref[...] = dsrc
        ddst_ref[...] = ddst
        h1 = jnp.dot(x_ref[...], w_ref[...], preferred_element_type=jnp.float32)
        h_ref[pl.ds(0, N), :] = h1 * dsrc[:N]
        h_ref[pl.ds(N, NPAD - N), :] = jnp.zeros((NPAD - N, D_HID), jnp.float32)

    return pl.pallas_call(
        body,
        out_shape=[
            jax.ShapeDtypeStruct((NPAD, D_HID), jnp.float32),
            jax.ShapeDtypeStruct((NPAD, 1), jnp.float32),
            jax.ShapeDtypeStruct((NPAD, 1), jnp.float32),
        ],
    )(degp4, x, w1)


def _tc_encode2(p, ddst, dsrc, w2):
    def body(p_ref, ddst_ref, dsrc_ref, w_ref, out_ref):
        agg = (p_ref[0] + p_ref[1]) * ddst_ref[...]
        h = jnp.maximum(agg, 0.0)
        out_ref[...] = jnp.dot(h, w_ref[...],
                               preferred_element_type=jnp.float32) * dsrc_ref[...]

    return pl.pallas_call(
        body,
        out_shape=jax.ShapeDtypeStruct((NPAD, D_OUT), jnp.float32),
    )(p, ddst, dsrc, w2)


def _tc_final(q, ddst):
    def body(q_ref, ddst_ref, out_ref):
        z = (q_ref[0] + q_ref[1]) * ddst_ref[...]
        out_ref[...] = z[:N]

    return pl.pallas_call(
        body,
        out_shape=jax.ShapeDtypeStruct((N, D_OUT), jnp.float32),
    )(q, ddst)


def kernel(x, adj, W1, W2):
    adj32 = adj.astype(jnp.int32)
    src = adj32[0]
    dst = adj32[1]
    pad = (jnp.arange(EPAD - E, dtype=jnp.int32) % (NPAD - N)) + N
    src2d = jnp.concatenate([src, pad]).reshape(NCHUNKS, C)
    dst2d = jnp.concatenate([dst, pad]).reshape(NCHUNKS, C)
    zrow = jnp.zeros((NPAD,), jnp.float32)
    z32 = jnp.zeros((NPAD, D_HID), jnp.float32)
    z16 = jnp.zeros((NPAD, D_OUT), jnp.float32)

    degp = _sc_degrees(src2d, dst2d, zrow)
    h1s, dsrc, ddst = _tc_encode1(degp.reshape(2, NC, NPAD, 1), x, W1)
    p1 = _sc_propagate(h1s, src2d, dst2d, z32, D_HID)
    h2s = _tc_encode2(p1, ddst, dsrc, W2)
    p2 = _sc_propagate(h2s, src2d, dst2d, z16, D_OUT)
    return _tc_final(p2, ddst)


# speedup vs baseline: 1.0519x; 1.0519x over previous
import functools

import jax
import jax.numpy as jnp
from jax import lax
from jax.experimental import pallas as pl
from jax.experimental.pallas import tpu as pltpu
from jax.experimental.pallas import tpu_sc as plsc

N = 10000
E = 320000
D_IN = 128
D_HID = 32
D_OUT = 16

NC = 2
NS = 16
NW = NC * NS

C = 128
NB = 10
TCH = 80
NCHUNKS = NW * TCH
EPAD = NCHUNKS * C
NPAD = 10240
RPT = NPAD // NS
NGRP = TCH // NB


def _mesh():
    return plsc.VectorSubcoreMesh(core_axis_name="c", subcore_axis_name="s")


def _sc_degrees(src2d, dst2d, zrow):
    @functools.partial(
        pl.kernel,
        out_type=jax.ShapeDtypeStruct((2, NC, NPAD), jnp.float32),
        mesh=_mesh(),
        scratch_types=[
            pltpu.VMEM((TCH, C), jnp.int32),
            pltpu.VMEM((C,), jnp.float32),
            pltpu.VMEM_SHARED((NPAD,), jnp.float32),
            pltpu.VMEM_SHARED((NPAD,), jnp.float32),
            pltpu.SemaphoreType.DMA((NB,)),
        ],
    )
    def deg_kernel(src_hbm, dst_hbm, z_hbm, out_hbm, idx_v, ones_v, acc_out,
                   acc_in, ssem):
        c = lax.axis_index("c")
        s = lax.axis_index("s")
        wid = c * NS + s
        for i in range(C // 16):
            ones_v[pl.ds(i * 16, 16)] = jnp.ones((16,), jnp.float32)
        pltpu.sync_copy(z_hbm.at[pl.ds(s * RPT, RPT)],
                        acc_out.at[pl.ds(s * RPT, RPT)])
        pltpu.sync_copy(z_hbm.at[pl.ds(s * RPT, RPT)],
                        acc_in.at[pl.ds(s * RPT, RPT)])
        plsc.subcore_barrier()

        for idx_hbm, acc in ((src_hbm, acc_out), (dst_hbm, acc_in)):
            pltpu.sync_copy(idx_hbm.at[pl.ds(wid * TCH, TCH)], idx_v)

            def sstart(j, bb, acc=acc):
                pltpu.async_copy(ones_v, acc.at[idx_v.at[j]], ssem.at[bb],
                                 add=True)

            def swait(j, bb, acc=acc):
                pltpu.make_async_copy(ones_v, acc.at[idx_v.at[j]],
                                      ssem.at[bb]).wait()

            for b in range(NB):
                sstart(b, b)

            @pl.loop(0, NGRP - 1)
            def _grp(g):
                for b in range(NB):
                    j = g * NB + b
                    swait(j, b)
                    sstart(j + NB, b)

            for b in range(NB):
                swait((NGRP - 1) * NB + b, b)

        plsc.subcore_barrier()
        pltpu.sync_copy(acc_out.at[pl.ds(s * RPT, RPT)],
                        out_hbm.at[0, c, pl.ds(s * RPT, RPT)])
        pltpu.sync_copy(acc_in.at[pl.ds(s * RPT, RPT)],
                        out_hbm.at[1, c, pl.ds(s * RPT, RPT)])

    return deg_kernel(src2d, dst2d, zrow)


def _sc_propagate(h, src2d, dst2d, zero_acc, d):
    @functools.partial(
        pl.kernel,
        out_type=jax.ShapeDtypeStruct((NC, NPAD, d), jnp.float32),
        mesh=_mesh(),
        compiler_params=pltpu.CompilerParams(use_tc_tiling_on_sc=False),
        scratch_types=[
            pltpu.VMEM((TCH, C), jnp.int32),
            pltpu.VMEM((TCH, C), jnp.int32),
            pltpu.VMEM((NB, C, d), jnp.float32),
            pltpu.VMEM_SHARED((NPAD, d), jnp.float32),
            pltpu.SemaphoreType.DMA((NB,)),
            pltpu.SemaphoreType.DMA((NB,)),
        ],
    )
    def prop_kernel(h_hbm, src_hbm, dst_hbm, z_hbm, out_hbm, src_idx, dst_idx,
                    rows, acc, gsem, ssem):
        c = lax.axis_index("c")
        s = lax.axis_index("s")
        wid = c * NS + s
        pltpu.sync_copy(src_hbm.at[pl.ds(wid * TCH, TCH)], src_idx)
        pltpu.sync_copy(dst_hbm.at[pl.ds(wid * TCH, TCH)], dst_idx)

        def gstart(j, bb):
            pltpu.async_copy(h_hbm.at[src_idx.at[j]], rows.at[bb], gsem.at[bb])

        def gwait(j, bb):
            pltpu.make_async_copy(h_hbm.at[src_idx.at[j]], rows.at[bb],
                                  gsem.at[bb]).wait()

        def sstart(j, bb):
            pltpu.async_copy(rows.at[bb], acc.at[dst_idx.at[j]], ssem.at[bb],
                             add=True)

        def swait(j, bb):
            pltpu.make_async_copy(rows.at[bb], acc.at[dst_idx.at[j]],
                                  ssem.at[bb]).wait()

        for b in range(NB):
            gstart(b, b)
        pltpu.sync_copy(z_hbm.at[pl.ds(s * RPT, RPT)],
                        acc.at[pl.ds(s * RPT, RPT)])
        plsc.subcore_barrier()

        @pl.loop(0, NGRP - 1)
        def _grp(g):
            for b in range(NB):
                j = g * NB + b
                gwait(j, b)
                sstart(j, b)
            for b in range(NB):
                j = g * NB + b
                swait(j, b)
                gstart(j + NB, b)

        for b in range(NB):
            j = (NGRP - 1) * NB + b
            gwait(j, b)
            sstart(j, b)
        for b in range(NB):
            swait((NGRP - 1) * NB + b, b)

        plsc.subcore_barrier()
        pltpu.sync_copy(acc.at[pl.ds(s * RPT, RPT)],
                        out_hbm.at[c, pl.ds(s * RPT, RPT)])

    return prop_kernel(h, src2d, dst2d, zero_acc)


def _dinv(deg):
    return jnp.where(deg > 0, lax.rsqrt(jnp.maximum(deg, 1e-12)), 0.0)


def _tc_encode1(degp4, x, w1):
    def body(degp_ref, x_ref, w_ref, h_ref, dsrc_ref, ddst_ref):
        dsrc = _dinv(degp_ref[0, 0] + degp_ref[0, 1])
        ddst = _dinv(degp_ref[1, 0] + degp_ref[1, 1])
        dsrc_ref[...] = dsrc
        ddst_ref[...] = ddst
        h1 = jnp.dot(x_ref[...], w_ref[...], preferred_element_type=jnp.float32)
        h_ref[pl.ds(0, N), :] = h1 * dsrc[:N]
        h_ref[pl.ds(N, NPAD - N), :] = jnp.zeros((NPAD - N, D_HID), jnp.float32)

    return pl.pallas_call(
        body,
        out_shape=[
            jax.ShapeDtypeStruct((NPAD, D_HID), jnp.float32),
            jax.ShapeDtypeStruct((NPAD, 1), jnp.float32),
            jax.ShapeDtypeStruct((NPAD, 1), jnp.float32),
        ],
    )(degp4, x, w1)


def _tc_encode2(p, ddst, dsrc, w2):
    def body(p_ref, ddst_ref, dsrc_ref, w_ref, out_ref):
        agg = (p_ref[0] + p_ref[1]) * ddst_ref[...]
        h = jnp.maximum(agg, 0.0)
        out_ref[...] = jnp.dot(h, w_ref[...],
                               preferred_element_type=jnp.float32) * dsrc_ref[...]

    return pl.pallas_call(
        body,
        out_shape=jax.ShapeDtypeStruct((NPAD, D_OUT), jnp.float32),
    )(p, ddst, dsrc, w2)


def _tc_final(q, ddst):
    def body(q_ref, ddst_ref, out_ref):
        z = (q_ref[0] + q_ref[1]) * ddst_ref[...]
        out_ref[...] = z[:N]

    return pl.pallas_call(
        body,
        out_shape=jax.ShapeDtypeStruct((N, D_OUT), jnp.float32),
    )(q, ddst)


def kernel(x, adj, W1, W2):
    adj32 = adj.astype(jnp.int32)
    src = adj32[0]
    dst = adj32[1]
    pad = (jnp.arange(EPAD - E, dtype=jnp.int32) % (NPAD - N)) + N
    src2d = jnp.concatenate([src, pad]).reshape(NCHUNKS, C)
    dst2d = jnp.concatenate([dst, pad]).reshape(NCHUNKS, C)
    zrow = jnp.zeros((NPAD,), jnp.float32)
    z32 = jnp.zeros((NPAD, D_HID), jnp.float32)
    z16 = jnp.zeros((NPAD, D_OUT), jnp.float32)

    degp = _sc_degrees(src2d, dst2d, zrow)
    h1s, dsrc, ddst = _tc_encode1(degp.reshape(2, NC, NPAD, 1), x, W1)
    p1 = _sc_propagate(h1s, src2d, dst2d, z32, D_HID)
    h2s = _tc_encode2(p1, ddst, dsrc, W2)
    p2 = _sc_propagate(h2s, src2d, dst2d, z16, D_OUT)
    return _tc_final(p2, ddst)


# speedup vs baseline: 1.2146x; 1.1546x over previous
import functools

import jax
import jax.numpy as jnp
from jax import lax
from jax.experimental import pallas as pl
from jax.experimental.pallas import tpu as pltpu
from jax.experimental.pallas import tpu_sc as plsc

N = 10000
E = 320000
D_IN = 128
D_HID = 32
D_OUT = 16

NC = 2
NS = 16
NW = NC * NS

C = 128
NB = 10
TCH = 80
NCHUNKS = NW * TCH
EPAD = NCHUNKS * C
NPAD = 10240
RPT = NPAD // NS
NGRP = TCH // NB


def _mesh():
    return plsc.VectorSubcoreMesh(core_axis_name="c", subcore_axis_name="s")


def _sc_degrees(src2d, dst2d, zrow):
    @functools.partial(
        pl.kernel,
        out_type=jax.ShapeDtypeStruct((2, NC, NPAD), jnp.float32),
        mesh=_mesh(),
        scratch_types=[
            pltpu.VMEM((TCH, C), jnp.int32),
            pltpu.VMEM((C,), jnp.float32),
            pltpu.VMEM_SHARED((NPAD,), jnp.float32),
            pltpu.VMEM_SHARED((NPAD,), jnp.float32),
            pltpu.SemaphoreType.DMA((NB,)),
        ],
    )
    def deg_kernel(src_hbm, dst_hbm, z_hbm, out_hbm, idx_v, ones_v, acc_out,
                   acc_in, ssem):
        c = lax.axis_index("c")
        s = lax.axis_index("s")
        wid = c * NS + s
        for i in range(C // 16):
            ones_v[pl.ds(i * 16, 16)] = jnp.ones((16,), jnp.float32)
        pltpu.sync_copy(z_hbm.at[pl.ds(s * RPT, RPT)],
                        acc_out.at[pl.ds(s * RPT, RPT)])
        pltpu.sync_copy(z_hbm.at[pl.ds(s * RPT, RPT)],
                        acc_in.at[pl.ds(s * RPT, RPT)])
        plsc.subcore_barrier()

        for idx_hbm, acc in ((src_hbm, acc_out), (dst_hbm, acc_in)):
            pltpu.sync_copy(idx_hbm.at[pl.ds(wid * TCH, TCH)], idx_v)

            def sstart(j, bb, acc=acc):
                pltpu.async_copy(ones_v, acc.at[idx_v.at[j]], ssem.at[bb],
                                 add=True)

            def swait(j, bb, acc=acc):
                pltpu.make_async_copy(ones_v, acc.at[idx_v.at[j]],
                                      ssem.at[bb]).wait()

            for b in range(NB):
                sstart(b, b)

            @pl.loop(0, NGRP - 1)
            def _grp(g):
                for b in range(NB):
                    j = g * NB + b
                    swait(j, b)
                    sstart(j + NB, b)

            for b in range(NB):
                swait((NGRP - 1) * NB + b, b)

        plsc.subcore_barrier()
        pltpu.sync_copy(acc_out.at[pl.ds(s * RPT, RPT)],
                        out_hbm.at[0, c, pl.ds(s * RPT, RPT)])
        pltpu.sync_copy(acc_in.at[pl.ds(s * RPT, RPT)],
                        out_hbm.at[1, c, pl.ds(s * RPT, RPT)])

    return deg_kernel(src2d, dst2d, zrow)


def _sc_propagate(h, src2d, dst2d, zero_acc, d):
    @functools.partial(
        pl.kernel,
        out_type=jax.ShapeDtypeStruct((NC, NPAD, d), jnp.float32),
        mesh=_mesh(),
        compiler_params=pltpu.CompilerParams(use_tc_tiling_on_sc=False),
        scratch_types=[
            pltpu.VMEM((TCH, C), jnp.int32),
            pltpu.VMEM((TCH, C), jnp.int32),
            pltpu.VMEM((NB, C, d), jnp.float32),
            pltpu.VMEM_SHARED((NPAD, d), jnp.float32),
            pltpu.SemaphoreType.DMA((NB,)),
            pltpu.SemaphoreType.DMA((NB,)),
        ],
    )
    def prop_kernel(h_hbm, src_hbm, dst_hbm, z_hbm, out_hbm, src_idx, dst_idx,
                    rows, acc, gsem, ssem):
        c = lax.axis_index("c")
        s = lax.axis_index("s")
        wid = c * NS + s
        pltpu.sync_copy(src_hbm.at[pl.ds(wid * TCH, TCH)], src_idx)
        pltpu.sync_copy(dst_hbm.at[pl.ds(wid * TCH, TCH)], dst_idx)

        def gstart(j, bb):
            pltpu.async_copy(h_hbm.at[src_idx.at[j]], rows.at[bb], gsem.at[bb])

        def gwait(j, bb):
            pltpu.make_async_copy(h_hbm.at[src_idx.at[j]], rows.at[bb],
                                  gsem.at[bb]).wait()

        def sstart(j, bb):
            pltpu.async_copy(rows.at[bb], acc.at[dst_idx.at[j]], ssem.at[bb],
                             add=True)

        def swait(j, bb):
            pltpu.make_async_copy(rows.at[bb], acc.at[dst_idx.at[j]],
                                  ssem.at[bb]).wait()

        for b in range(NB):
            gstart(b, b)
        pltpu.sync_copy(z_hbm.at[pl.ds(s * RPT, RPT)],
                        acc.at[pl.ds(s * RPT, RPT)])
        plsc.subcore_barrier()

        @pl.loop(0, NGRP - 1)
        def _grp(g):
            for b in range(NB):
                j = g * NB + b
                gwait(j, b)
                sstart(j, b)
            for b in range(NB):
                j = g * NB + b
                swait(j, b)
                gstart(j + NB, b)

        for b in range(NB):
            j = (NGRP - 1) * NB + b
            gwait(j, b)
            sstart(j, b)
        for b in range(NB):
            swait((NGRP - 1) * NB + b, b)

        plsc.subcore_barrier()
        pltpu.sync_copy(acc.at[pl.ds(s * RPT, RPT)],
                        out_hbm.at[c, pl.ds(s * RPT, RPT)])

    return prop_kernel(h, src2d, dst2d, zero_acc)


def _dinv(deg):
    return jnp.where(deg > 0, lax.rsqrt(jnp.maximum(deg, 1e-12)), 0.0)


def _dsrc_col(degp_ref):
    return _dinv(degp_ref[0, 0] + degp_ref[0, 1]).reshape(NPAD, 1)


def _ddst_col(degp_ref):
    return _dinv(degp_ref[1, 0] + degp_ref[1, 1]).reshape(NPAD, 1)


def _tc_encode1(degp, x, w1):
    def body(degp_ref, x_ref, w_ref, h_ref):
        dsrc = _dsrc_col(degp_ref)
        h1 = jnp.dot(x_ref[...], w_ref[...], preferred_element_type=jnp.float32)
        h_ref[pl.ds(0, N), :] = h1 * dsrc[:N]
        h_ref[pl.ds(N, NPAD - N), :] = jnp.zeros((NPAD - N, D_HID), jnp.float32)

    return pl.pallas_call(
        body,
        out_shape=jax.ShapeDtypeStruct((NPAD, D_HID), jnp.float32),
    )(degp, x, w1)


def _tc_encode2(p, degp, w2):
    def body(p_ref, degp_ref, w_ref, out_ref):
        agg = (p_ref[0] + p_ref[1]) * _ddst_col(degp_ref)
        h = jnp.maximum(agg, 0.0)
        out_ref[...] = jnp.dot(h, w_ref[...],
                               preferred_element_type=jnp.float32) * _dsrc_col(degp_ref)

    return pl.pallas_call(
        body,
        out_shape=jax.ShapeDtypeStruct((NPAD, D_OUT), jnp.float32),
    )(p, degp, w2)


def _tc_final(q, degp):
    def body(q_ref, degp_ref, out_ref):
        z = (q_ref[0] + q_ref[1]) * _ddst_col(degp_ref)
        out_ref[...] = z[:N]

    return pl.pallas_call(
        body,
        out_shape=jax.ShapeDtypeStruct((N, D_OUT), jnp.float32),
    )(q, degp)


def kernel(x, adj, W1, W2):
    adj32 = adj.astype(jnp.int32)
    src = adj32[0]
    dst = adj32[1]
    pad = (jnp.arange(EPAD - E, dtype=jnp.int32) % (NPAD - N)) + N
    src2d = jnp.concatenate([src, pad]).reshape(NCHUNKS, C)
    dst2d = jnp.concatenate([dst, pad]).reshape(NCHUNKS, C)
    zrow = jnp.zeros((NPAD,), jnp.float32)
    z32 = jnp.zeros((NPAD, D_HID), jnp.float32)
    z16 = jnp.zeros((NPAD, D_OUT), jnp.float32)

    degp = _sc_degrees(src2d, dst2d, zrow)
    h1s = _tc_encode1(degp, x, W1)
    p1 = _sc_propagate(h1s, src2d, dst2d, z32, D_HID)
    h2s = _tc_encode2(p1, degp, W2)
    p2 = _sc_propagate(h2s, src2d, dst2d, z16, D_OUT)
    return _tc_final(p2, degp)


# speedup vs baseline: 1.2177x; 1.0026x over previous
import functools

import jax
import jax.numpy as jnp
from jax import lax
from jax.experimental import pallas as pl
from jax.experimental.pallas import tpu as pltpu
from jax.experimental.pallas import tpu_sc as plsc

N = 10000
E = 320000
D_IN = 128
D_HID = 32
D_OUT = 16

NC = 2
NS = 16
NW = NC * NS

C = 128
NB = 10
TCH = 80
NCHUNKS = NW * TCH
EPAD = NCHUNKS * C
NPAD = 10240
RPT = NPAD // NS
NGRP = TCH // NB


def _mesh():
    return plsc.VectorSubcoreMesh(core_axis_name="c", subcore_axis_name="s")


def _sc_degrees(src2d, dst2d, zrow):
    @functools.partial(
        pl.kernel,
        out_type=jax.ShapeDtypeStruct((2, NC, NPAD), jnp.float32),
        mesh=_mesh(),
        scratch_types=[
            pltpu.VMEM((TCH, C), jnp.int32),
            pltpu.VMEM((C,), jnp.float32),
            pltpu.VMEM_SHARED((NPAD,), jnp.float32),
            pltpu.VMEM_SHARED((NPAD,), jnp.float32),
            pltpu.SemaphoreType.DMA((NB,)),
        ],
    )
    def deg_kernel(src_hbm, dst_hbm, z_hbm, out_hbm, idx_v, ones_v, acc_out,
                   acc_in, ssem):
        c = lax.axis_index("c")
        s = lax.axis_index("s")
        wid = c * NS + s
        for i in range(C // 16):
            ones_v[pl.ds(i * 16, 16)] = jnp.ones((16,), jnp.float32)
        pltpu.sync_copy(z_hbm.at[pl.ds(s * RPT, RPT)],
                        acc_out.at[pl.ds(s * RPT, RPT)])
        pltpu.sync_copy(z_hbm.at[pl.ds(s * RPT, RPT)],
                        acc_in.at[pl.ds(s * RPT, RPT)])
        plsc.subcore_barrier()

        for idx_hbm, acc in ((src_hbm, acc_out), (dst_hbm, acc_in)):
            pltpu.sync_copy(idx_hbm.at[pl.ds(wid * TCH, TCH)], idx_v)

            def sstart(j, bb, acc=acc):
                pltpu.async_copy(ones_v, acc.at[idx_v.at[j]], ssem.at[bb],
                                 add=True)

            def swait(j, bb, acc=acc):
                pltpu.make_async_copy(ones_v, acc.at[idx_v.at[j]],
                                      ssem.at[bb]).wait()

            for b in range(NB):
                sstart(b, b)

            @pl.loop(0, NGRP - 1)
            def _grp(g):
                for b in range(NB):
                    j = g * NB + b
                    swait(j, b)
                    sstart(j + NB, b)

            for b in range(NB):
                swait((NGRP - 1) * NB + b, b)

        plsc.subcore_barrier()
        pltpu.sync_copy(acc_out.at[pl.ds(s * RPT, RPT)],
                        out_hbm.at[0, c, pl.ds(s * RPT, RPT)])
        pltpu.sync_copy(acc_in.at[pl.ds(s * RPT, RPT)],
                        out_hbm.at[1, c, pl.ds(s * RPT, RPT)])

    return deg_kernel(src2d, dst2d, zrow)


def _sc_propagate(h, src2d, dst2d, zero_acc, d):
    @functools.partial(
        pl.kernel,
        out_type=jax.ShapeDtypeStruct((NC, NPAD, d), jnp.float32),
        mesh=_mesh(),
        compiler_params=pltpu.CompilerParams(use_tc_tiling_on_sc=False),
        scratch_types=[
            pltpu.VMEM((TCH, C), jnp.int32),
            pltpu.VMEM((TCH, C), jnp.int32),
            pltpu.VMEM((NB, C, d), jnp.float32),
            pltpu.VMEM_SHARED((NPAD, d), jnp.float32),
            pltpu.SemaphoreType.DMA((NB,)),
            pltpu.SemaphoreType.DMA((NB,)),
        ],
    )
    def prop_kernel(h_hbm, src_hbm, dst_hbm, z_hbm, out_hbm, src_idx, dst_idx,
                    rows, acc, gsem, ssem):
        c = lax.axis_index("c")
        s = lax.axis_index("s")
        wid = c * NS + s
        pltpu.sync_copy(src_hbm.at[pl.ds(wid * TCH, TCH)], src_idx)
        pltpu.sync_copy(dst_hbm.at[pl.ds(wid * TCH, TCH)], dst_idx)

        def gstart(j, bb):
            pltpu.async_copy(h_hbm.at[src_idx.at[j]], rows.at[bb], gsem.at[bb])

        def gwait(j, bb):
            pltpu.make_async_copy(h_hbm.at[src_idx.at[j]], rows.at[bb],
                                  gsem.at[bb]).wait()

        def sstart(j, bb):
            pltpu.async_copy(rows.at[bb], acc.at[dst_idx.at[j]], ssem.at[bb],
                             add=True)

        def swait(j, bb):
            pltpu.make_async_copy(rows.at[bb], acc.at[dst_idx.at[j]],
                                  ssem.at[bb]).wait()

        for b in range(NB):
            gstart(b, b)
        pltpu.sync_copy(z_hbm.at[pl.ds(s * RPT, RPT)],
                        acc.at[pl.ds(s * RPT, RPT)])
        plsc.subcore_barrier()

        @pl.loop(0, NGRP - 1)
        def _grp(g):
            for b in range(NB):
                j = g * NB + b
                gwait(j, b)
                sstart(j, b)
            for b in range(NB):
                j = g * NB + b
                swait(j, b)
                gstart(j + NB, b)

        for b in range(NB):
            j = (NGRP - 1) * NB + b
            gwait(j, b)
            sstart(j, b)
        for b in range(NB):
            swait((NGRP - 1) * NB + b, b)

        plsc.subcore_barrier()
        pltpu.sync_copy(acc.at[pl.ds(s * RPT, RPT)],
                        out_hbm.at[c, pl.ds(s * RPT, RPT)])

    return prop_kernel(h, src2d, dst2d, zero_acc)


def _dinv(deg):
    return jnp.where(deg > 0, lax.rsqrt(jnp.maximum(deg, 1e-12)), 0.0)


def _dsrc_col(degp_ref):
    return _dinv(degp_ref[0, 0] + degp_ref[0, 1]).reshape(NPAD, 1)


def _ddst_col(degp_ref):
    return _dinv(degp_ref[1, 0] + degp_ref[1, 1]).reshape(NPAD, 1)


def _tc_matmul1(x, w1):
    def body(x_ref, w_ref, h_ref):
        h_ref[...] = jnp.dot(x_ref[...], w_ref[...],
                             preferred_element_type=jnp.float32)

    return pl.pallas_call(
        body,
        grid=(5,),
        in_specs=[
            pl.BlockSpec((N // 5, D_IN), lambda i: (i, 0)),
            pl.BlockSpec((D_IN, D_HID), lambda i: (0, 0)),
        ],
        out_specs=pl.BlockSpec((N // 5, D_HID), lambda i: (i, 0)),
        out_shape=jax.ShapeDtypeStruct((N, D_HID), jnp.float32),
    )(x, w1)


def _tc_scale1(h1, degp):
    def body(h1_ref, degp_ref, h_ref):
        dsrc = _dsrc_col(degp_ref)
        h_ref[pl.ds(0, N), :] = h1_ref[...] * dsrc[:N]
        h_ref[pl.ds(N, NPAD - N), :] = jnp.zeros((NPAD - N, D_HID), jnp.float32)

    return pl.pallas_call(
        body,
        out_shape=jax.ShapeDtypeStruct((NPAD, D_HID), jnp.float32),
    )(h1, degp)


def _tc_encode2(p, degp, w2):
    def body(p_ref, degp_ref, w_ref, out_ref):
        agg = (p_ref[0] + p_ref[1]) * _ddst_col(degp_ref)
        h = jnp.maximum(agg, 0.0)
        out_ref[...] = jnp.dot(h, w_ref[...],
                               preferred_element_type=jnp.float32) * _dsrc_col(degp_ref)

    return pl.pallas_call(
        body,
        out_shape=jax.ShapeDtypeStruct((NPAD, D_OUT), jnp.float32),
    )(p, degp, w2)


def _tc_final(q, degp):
    def body(q_ref, degp_ref, out_ref):
        z = (q_ref[0] + q_ref[1]) * _ddst_col(degp_ref)
        out_ref[...] = z[:N]

    return pl.pallas_call(
        body,
        out_shape=jax.ShapeDtypeStruct((N, D_OUT), jnp.float32),
    )(q, degp)


def kernel(x, adj, W1, W2):
    adj32 = adj.astype(jnp.int32)
    src = adj32[0]
    dst = adj32[1]
    pad = (jnp.arange(EPAD - E, dtype=jnp.int32) % (NPAD - N)) + N
    src2d = jnp.concatenate([src, pad]).reshape(NCHUNKS, C)
    dst2d = jnp.concatenate([dst, pad]).reshape(NCHUNKS, C)
    zrow = jnp.zeros((NPAD,), jnp.float32)
    z32 = jnp.zeros((NPAD, D_HID), jnp.float32)
    z16 = jnp.zeros((NPAD, D_OUT), jnp.float32)

    h1 = _tc_matmul1(x, W1)
    degp = _sc_degrees(src2d, dst2d, zrow)
    h1s = _tc_scale1(h1, degp)
    p1 = _sc_propagate(h1s, src2d, dst2d, z32, D_HID)
    h2s = _tc_encode2(p1, degp, W2)
    p2 = _sc_propagate(h2s, src2d, dst2d, z16, D_OUT)
    return _tc_final(p2, degp)
